# baseline (speedup 1.0000x reference)
import jax
import jax.numpy as jnp
from jax import lax
from jax.experimental import pallas as pl
from jax.experimental.pallas import tpu as pltpu
from jax.experimental.pallas import tpu_sc as plsc

MS = 512
NB = 64
N = MS * (MS + 1) // 2
RBLK = 32
NBLK = MS // RBLK
LANE = 16
NCH = MS // LANE
DIAG = RBLK // LANE
BSLAB = 16
NQ = NB // (2 * BSLAB)


def _offset(r: int) -> int:
    return MS * r - r * (r - 1) // 2


_ALO = []
_SPAN = []
for _c in range(NBLK):
    _r0 = _c * RBLK
    _r1 = _r0 + RBLK - 1
    _lo = max(_offset(_r0) - _r0, 0) & ~127
    _hi = _offset(_r1) + (MS - _r1)
    _span = -(-(_hi - _lo) // 128) * 128
    _ALO.append(_lo)
    _SPAN.append(_span)
    assert _lo + _span <= N
_VIN = max(_SPAN)


def _body(x_hbm, out_hbm, vin0, vin1, vout0, vout1,
          slab0, slab1, slab2, slab3,
          slab_sem, fill_sem, out_sem0, out_sem1):
    cid = lax.axis_index("c")
    sid = lax.axis_index("s")

    lane = lax.iota(jnp.int32, LANE)
    vins = [vin0, vin1]
    vouts = [vout0, vout1]
    out_sems = [out_sem0, out_sem1]
    slabs = [slab0, slab1, slab2, slab3]

    def slab_copy(q, c):
        b0 = cid * (NB // 2) + q * BSLAB
        return pltpu.make_async_copy(
            x_hbm.at[pl.ds(b0, BSLAB), pl.ds(_ALO[c], _SPAN[c])],
            slabs[c % 4].at[:, pl.ds(0, _SPAN[c])], slab_sem)

    def fill_copy(c):
        return pltpu.make_async_copy(
            slabs[c % 4].at[sid, pl.ds(0, _SPAN[c])],
            vins[c % 2].at[pl.ds(0, _SPAN[c])], fill_sem)

    def out_copy(q, c):
        b = cid * (NB // 2) + q * BSLAB + sid
        return pltpu.make_async_copy(
            vouts[c % 2], out_hbm.at[b, pl.ds(c * RBLK, RBLK)],
            out_sems[c % 2])

    slab_copy(0, 0).start()
    slab_copy(0, 1).start()
    slab_copy(0, 0).wait()
    slab_copy(0, 1).wait()
    plsc.subcore_barrier()
    fill_copy(0).start()
    fill_copy(0).wait()

    def q_body(q, _):
        for c in range(NBLK):
            r0 = c * RBLK
            alo = _ALO[c]
            vin = vins[c % 2]
            vout = vouts[c % 2]

            if c + 2 < NBLK:
                slab_copy(q, c + 2).start()
            else:
                @pl.when(q < NQ - 1)
                def _(q=q, c=c):
                    slab_copy(q + 1, c + 2 - NBLK).start()
            if c + 1 < NBLK:
                fill_copy(c + 1).start()
            else:
                @pl.when(q < NQ - 1)
                def _(q=q):
                    fill_copy(0).start()

            if c >= 2:
                out_copy(q, c - 2).wait()
            else:
                @pl.when(q > 0)
                def _(q=q, c=c):
                    out_copy(q - 1, c + NBLK - 2).wait()

            @plsc.parallel_loop(0, RBLK, 1, unroll=1)
            def row(i, c=c, r0=r0, alo=alo, vin=vin, vout=vout):
                r = r0 + i
                base = (MS * r - ((r * (r - 1)) >> 1)) - r - alo
                zeros = jnp.zeros((LANE,), jnp.float32)
                for k in range(max(0, DIAG * (c - 2)), NCH):
                    if k < DIAG * c:
                        vout[i, pl.ds(k * LANE, LANE)] = zeros
                    elif k < DIAG * (c + 1):
                        col = lane + (k * LANE)
                        keep = col >= r
                        g = plsc.load_gather(vin, [base + col], mask=keep)
                        vout[i, pl.ds(k * LANE, LANE)] = jnp.where(keep, g, 0.0)
                    else:
                        col = lane + (k * LANE)
                        g = plsc.load_gather(vin, [base + col])
                        vout[i, pl.ds(k * LANE, LANE)] = g

            out_copy(q, c).start()

            if c + 1 < NBLK:
                fill_copy(c + 1).wait()
            else:
                @pl.when(q < NQ - 1)
                def _(q=q):
                    fill_copy(0).wait()
            if c + 2 < NBLK:
                slab_copy(q, c + 2).wait()
            else:
                @pl.when(q < NQ - 1)
                def _(q=q, c=c):
                    slab_copy(q + 1, c + 2 - NBLK).wait()
            plsc.subcore_barrier()
        return 0

    lax.fori_loop(0, NQ, q_body, 0)

    out_copy(NQ - 1, NBLK - 2).wait()
    out_copy(NQ - 1, NBLK - 1).wait()


def kernel(inputs):
    sc_kernel = pl.kernel(
        _body,
        out_type=jax.ShapeDtypeStruct((NB, MS, MS), jnp.float32),
        mesh=plsc.VectorSubcoreMesh(core_axis_name="c", subcore_axis_name="s"),
        scratch_types=[
            pltpu.VMEM((_VIN,), jnp.float32),
            pltpu.VMEM((_VIN,), jnp.float32),
            pltpu.VMEM((RBLK, MS), jnp.float32),
            pltpu.VMEM((RBLK, MS), jnp.float32),
            pltpu.VMEM_SHARED((BSLAB, _VIN), jnp.float32),
            pltpu.VMEM_SHARED((BSLAB, _VIN), jnp.float32),
            pltpu.VMEM_SHARED((BSLAB, _VIN), jnp.float32),
            pltpu.VMEM_SHARED((BSLAB, _VIN), jnp.float32),
            pltpu.SemaphoreType.DMA,
            pltpu.SemaphoreType.DMA,
            pltpu.SemaphoreType.DMA,
            pltpu.SemaphoreType.DMA,
        ],
        compiler_params=pltpu.CompilerParams(needs_layout_passes=False),
    )
    return sc_kernel(inputs)


# speedup vs baseline: 5.3329x; 5.3329x over previous
import jax
import jax.numpy as jnp
from jax import lax
from jax.experimental import pallas as pl
from jax.experimental.pallas import tpu as pltpu
from jax.experimental.pallas import tpu_sc as plsc

MS = 512
NB = 64
N = MS * (MS + 1) // 2
RBLK = 32
NBLK = MS // RBLK
LANE = 16
NCH = MS // LANE
DIAG = RBLK // LANE
BSLAB = 16
NQ = NB // (2 * BSLAB)


def _offset(r: int) -> int:
    return MS * r - r * (r - 1) // 2


_ALO = []
_SPAN = []
for _c in range(NBLK):
    _r0 = _c * RBLK
    _r1 = _r0 + RBLK - 1
    _lo = max(_offset(_r0) - _r0, 0) & ~127
    _hi = _offset(_r1) + (MS - _r1)
    _span = -(-(_hi - _lo) // 128) * 128
    _ALO.append(_lo)
    _SPAN.append(_span)
    assert _lo + _span <= N
_VIN = max(_SPAN)


def _body(x_hbm, out_hbm, vin0, vin1, vout0, vout1,
          slab0, slab1, slab2, slab3,
          slab_sem, fill_sem, out_sem0, out_sem1):
    cid = lax.axis_index("c")
    sid = lax.axis_index("s")

    lane = lax.iota(jnp.int32, LANE)
    vins = [vin0, vin1]
    vouts = [vout0, vout1]
    out_sems = [out_sem0, out_sem1]
    slabs = [slab0, slab1, slab2, slab3]

    def slab_copy(q, c):
        b0 = cid * (NB // 2) + q * BSLAB
        return pltpu.make_async_copy(
            x_hbm.at[pl.ds(b0, BSLAB), pl.ds(_ALO[c], _SPAN[c])],
            slabs[c % 4].at[:, pl.ds(0, _SPAN[c])], slab_sem)

    def fill_copy(c):
        return pltpu.make_async_copy(
            slabs[c % 4].at[sid, pl.ds(0, _SPAN[c])],
            vins[c % 2].at[pl.ds(0, _SPAN[c])], fill_sem)

    def out_copy(q, c):
        b = cid * (NB // 2) + q * BSLAB + sid
        return pltpu.make_async_copy(
            vouts[c % 2], out_hbm.at[b, pl.ds(c * RBLK, RBLK)],
            out_sems[c % 2])

    @pl.when(sid == 0)
    def _():
        slab_copy(0, 0).start()
        slab_copy(0, 1).start()
        slab_copy(0, 0).wait()
        slab_copy(0, 1).wait()

    plsc.subcore_barrier()
    fill_copy(0).start()
    fill_copy(0).wait()

    def q_body(q, _):
        for c in range(NBLK):
            r0 = c * RBLK
            alo = _ALO[c]
            vin = vins[c % 2]
            vout = vouts[c % 2]

            if c + 2 < NBLK:
                @pl.when(sid == 0)
                def _(q=q, c=c):
                    slab_copy(q, c + 2).start()
            else:
                @pl.when(jnp.logical_and(sid == 0, q < NQ - 1))
                def _(q=q, c=c):
                    slab_copy(q + 1, c + 2 - NBLK).start()
            if c + 1 < NBLK:
                fill_copy(c + 1).start()
            else:
                @pl.when(q < NQ - 1)
                def _(q=q):
                    fill_copy(0).start()

            if c >= 2:
                out_copy(q, c - 2).wait()
            else:
                @pl.when(q > 0)
                def _(q=q, c=c):
                    out_copy(q - 1, c + NBLK - 2).wait()

            @plsc.parallel_loop(0, RBLK, 1, unroll=1)
            def row(i, c=c, r0=r0, alo=alo, vin=vin, vout=vout):
                r = r0 + i
                base = (MS * r - ((r * (r - 1)) >> 1)) - r - alo
                zeros = jnp.zeros((LANE,), jnp.float32)
                for k in range(max(0, DIAG * (c - 2)), NCH):
                    if k < DIAG * c:
                        vout[i, pl.ds(k * LANE, LANE)] = zeros
                    elif k < DIAG * (c + 1):
                        col = lane + (k * LANE)
                        keep = col >= r
                        g = plsc.load_gather(vin, [base + col], mask=keep)
                        vout[i, pl.ds(k * LANE, LANE)] = jnp.where(keep, g, 0.0)
                    else:
                        col = lane + (k * LANE)
                        g = plsc.load_gather(vin, [base + col])
                        vout[i, pl.ds(k * LANE, LANE)] = g

            out_copy(q, c).start()

            if c + 1 < NBLK:
                fill_copy(c + 1).wait()
            else:
                @pl.when(q < NQ - 1)
                def _(q=q):
                    fill_copy(0).wait()
            if c + 2 < NBLK:
                @pl.when(sid == 0)
                def _(q=q, c=c):
                    slab_copy(q, c + 2).wait()
            else:
                @pl.when(jnp.logical_and(sid == 0, q < NQ - 1))
                def _(q=q, c=c):
                    slab_copy(q + 1, c + 2 - NBLK).wait()
            plsc.subcore_barrier()
        return 0

    lax.fori_loop(0, NQ, q_body, 0)

    out_copy(NQ - 1, NBLK - 2).wait()
    out_copy(NQ - 1, NBLK - 1).wait()


def kernel(inputs):
    sc_kernel = pl.kernel(
        _body,
        out_type=jax.ShapeDtypeStruct((NB, MS, MS), jnp.float32),
        mesh=plsc.VectorSubcoreMesh(core_axis_name="c", subcore_axis_name="s"),
        scratch_types=[
            pltpu.VMEM((_VIN,), jnp.float32),
            pltpu.VMEM((_VIN,), jnp.float32),
            pltpu.VMEM((RBLK, MS), jnp.float32),
            pltpu.VMEM((RBLK, MS), jnp.float32),
            pltpu.VMEM_SHARED((BSLAB, _VIN), jnp.float32),
            pltpu.VMEM_SHARED((BSLAB, _VIN), jnp.float32),
            pltpu.VMEM_SHARED((BSLAB, _VIN), jnp.float32),
            pltpu.VMEM_SHARED((BSLAB, _VIN), jnp.float32),
            pltpu.SemaphoreType.DMA,
            pltpu.SemaphoreType.DMA,
            pltpu.SemaphoreType.DMA,
            pltpu.SemaphoreType.DMA,
        ],
        compiler_params=pltpu.CompilerParams(needs_layout_passes=False),
    )
    return sc_kernel(inputs)
